# Initial kernel scaffold; baseline (speedup 1.0000x reference)
import jax
import jax.numpy as jnp
from jax.experimental import pallas as pl


def kernel(X, W_head, b_head, W_tail, b_tail):
    raise NotImplementedError("write your pallas kernel here")


# speedup vs baseline: 4.2110x; 4.2110x over previous
import jax
import jax.numpy as jnp
from jax.experimental import pallas as pl
from jax.experimental.pallas import tpu as pltpu

_DIM = 512
_N = 10000
_NP = 10240
_K = 32
_BR = 256
_CW = 1024
_NEG = -3.0e38
_BIGI = 2 ** 30


def _proj_kernel(x_ref, wh_ref, bh_ref, wt_ref, bt_ref, eh_ref, et_ref):
    x = x_ref[...]
    eh_ref[...] = jax.lax.dot_general(
        x, wh_ref[...], (((1,), (1,)), ((), ())),
        preferred_element_type=jnp.float32) + bh_ref[...]
    et_ref[...] = jax.lax.dot_general(
        x, wt_ref[...], (((1,), (1,)), ((), ())),
        preferred_element_type=jnp.float32) + bt_ref[...]


def _topk_kernel(eh_ref, et_ref, idx_ref, w_ref, l_ref):
    scale = _DIM ** -0.5
    eh = eh_ref[...]
    n_chunks = _NP // _CW
    cci = jax.lax.broadcasted_iota(jnp.int32, (_BR, _CW), 1)

    def mm_body(c, _):
        et = et_ref[pl.ds(c * _CW, _CW), :]
        prod = jax.lax.dot_general(
            eh, et, (((1,), (1,)), ((), ())),
            preferred_element_type=jnp.float32) * scale
        prod = jnp.where(c * _CW + cci < _N, prod, _NEG)
        l_ref[:, pl.ds(c * _CW, _CW)] = prod
        return 0

    jax.lax.fori_loop(0, n_chunks, mm_body, 0)

    ci = jax.lax.broadcasted_iota(jnp.int32, (_BR, _NP), 1)
    ki = jax.lax.broadcasted_iota(jnp.int32, (_BR, _K), 1)

    def sel_body(j, carry):
        vals, idxs = carry
        logits = l_ref[...]
        m = jnp.max(logits, axis=1, keepdims=True)
        hit = logits == m
        idx = jnp.min(jnp.where(hit, ci, _BIGI), axis=1, keepdims=True)
        l_ref[...] = jnp.where(ci == idx, _NEG, logits)
        vals = jnp.where(ki == j, m, vals)
        idxs = jnp.where(ki == j, idx, idxs)
        return vals, idxs

    vals = jnp.zeros((_BR, _K), jnp.float32)
    idxs = jnp.zeros((_BR, _K), jnp.int32)
    vals, idxs = jax.lax.fori_loop(0, _K, sel_body, (vals, idxs))

    e = jnp.exp(vals - vals[:, 0:1])
    w_ref[...] = e / jnp.sum(e, axis=1, keepdims=True)
    idx_ref[...] = idxs


def _build(X, W_head, b_head, W_tail, b_tail):
    Xp = jnp.pad(X, ((0, _NP - _N), (0, 0)))
    pb = _NP // _CW
    eh, et = pl.pallas_call(
        _proj_kernel,
        grid=(pb,),
        in_specs=[
            pl.BlockSpec((_CW, _DIM), lambda i: (i, 0)),
            pl.BlockSpec((_DIM, _DIM), lambda i: (0, 0)),
            pl.BlockSpec((1, _DIM), lambda i: (0, 0)),
            pl.BlockSpec((_DIM, _DIM), lambda i: (0, 0)),
            pl.BlockSpec((1, _DIM), lambda i: (0, 0)),
        ],
        out_specs=[
            pl.BlockSpec((_CW, _DIM), lambda i: (i, 0)),
            pl.BlockSpec((_CW, _DIM), lambda i: (i, 0)),
        ],
        out_shape=[
            jax.ShapeDtypeStruct((_NP, _DIM), jnp.float32),
            jax.ShapeDtypeStruct((_NP, _DIM), jnp.float32),
        ],
    )(Xp, W_head.reshape(_DIM, _DIM), b_head.reshape(1, _DIM),
      W_tail.reshape(_DIM, _DIM), b_tail.reshape(1, _DIM))

    nblk = _NP // _BR
    idxs, ws = pl.pallas_call(
        _topk_kernel,
        grid=(nblk,),
        in_specs=[
            pl.BlockSpec((_BR, _DIM), lambda i: (i, 0)),
            pl.BlockSpec((_NP, _DIM), lambda i: (0, 0)),
        ],
        out_specs=[
            pl.BlockSpec((_BR, _K), lambda i: (i, 0)),
            pl.BlockSpec((_BR, _K), lambda i: (i, 0)),
        ],
        out_shape=[
            jax.ShapeDtypeStruct((_NP, _K), jnp.int32),
            jax.ShapeDtypeStruct((_NP, _K), jnp.float32),
        ],
        scratch_shapes=[pltpu.VMEM((_BR, _NP), jnp.float32)],
    )(eh, et)
    return idxs[:_N], ws[:_N]


def kernel(X, W_head, b_head, W_tail, b_tail):
    idxs, ws = _build(X, W_head, b_head, W_tail, b_tail)
    n = X.shape[0]
    src = jnp.repeat(jnp.arange(n, dtype=jnp.int32), _K)
    edge_index = jnp.stack([src, idxs.reshape(-1)], axis=0)
    return edge_index, ws.reshape(-1)


# speedup vs baseline: 6.9831x; 1.6583x over previous
import jax
import jax.numpy as jnp
from jax.experimental import pallas as pl
from jax.experimental.pallas import tpu as pltpu

_DIM = 512
_N = 10000
_NP = 10240
_K = 32
_BR = 256
_CW = 1024
_NEG = -3.0e38
_BIGI = 2 ** 30


def _proj_kernel(x_ref, wh_ref, bh_ref, wt_ref, bt_ref, eh_ref, et_ref):
    x = x_ref[...]
    eh_ref[...] = jax.lax.dot_general(
        x, wh_ref[...], (((1,), (1,)), ((), ())),
        preferred_element_type=jnp.float32) + bh_ref[...]
    et_ref[...] = jax.lax.dot_general(
        x, wt_ref[...], (((1,), (1,)), ((), ())),
        preferred_element_type=jnp.float32) + bt_ref[...]


_D = 6


def _topk_kernel(eh_ref, et_ref, idx_ref, w_ref, l_ref):
    scale = _DIM ** -0.5
    eh = eh_ref[...]
    n_chunks = _NP // _CW
    n_groups = _NP // 128
    cci = jax.lax.broadcasted_iota(jnp.int32, (_BR, _CW), 1)

    def mm_body(c, _):
        et = et_ref[pl.ds(c * _CW, _CW), :]
        prod = jax.lax.dot_general(
            eh, et, (((1,), (1,)), ((), ())),
            preferred_element_type=jnp.float32) * scale
        prod = jnp.where(c * _CW + cci < _N, prod, _NEG)
        l_ref[:, pl.ds(c * _CW, _CW)] = prod
        return 0

    jax.lax.fori_loop(0, n_chunks, mm_body, 0)

    ki = jax.lax.broadcasted_iota(jnp.int32, (_BR, _K), 1)
    lane = jax.lax.broadcasted_iota(jnp.int32, (_BR, 128), 1)

    def ins_body(g, carry):
        s = list(carry[:_D])
        a = list(carry[_D:])
        x = l_ref[:, pl.ds(g * 128, 128)]
        ax = jnp.full((_BR, 128), 0, jnp.int32) + g
        for d in range(_D):
            c = x > s[d]
            s[d], x = jnp.where(c, x, s[d]), jnp.where(c, s[d], x)
            a[d], ax = jnp.where(c, ax, a[d]), jnp.where(c, a[d], ax)
        return tuple(s) + tuple(a)

    init = tuple(jnp.full((_BR, 128), _NEG, jnp.float32) for _ in range(_D)) \
        + tuple(jnp.zeros((_BR, 128), jnp.int32) for _ in range(_D))
    carry = jax.lax.fori_loop(0, n_groups, ins_body, init)
    s = carry[:_D]
    a = carry[_D:]

    def sel_body(j, carry):
        vals, idxs, cur, acur, p, of = carry
        m = jnp.max(cur, axis=1, keepdims=True)
        col = acur * 128 + lane
        hit = cur == m
        idx = jnp.min(jnp.where(hit, col, _BIGI), axis=1, keepdims=True)
        win = hit & (col == idx)
        pn = p + win.astype(jnp.int32)
        nv = jnp.full((_BR, 128), _NEG, jnp.float32)
        na = jnp.zeros((_BR, 128), jnp.int32)
        for d in range(1, _D):
            sel = pn == d
            nv = jnp.where(sel, s[d], nv)
            na = jnp.where(sel, a[d], na)
        cur = jnp.where(win, nv, cur)
        acur = jnp.where(win, na, acur)
        of = jnp.maximum(of, jnp.max((win & (pn >= _D)).astype(jnp.int32)))
        vals = jnp.where(ki == j, m, vals)
        idxs = jnp.where(ki == j, idx, idxs)
        return vals, idxs, cur, acur, pn, of

    vals = jnp.zeros((_BR, _K), jnp.float32)
    idxs = jnp.zeros((_BR, _K), jnp.int32)
    p0 = jnp.zeros((_BR, 128), jnp.int32)
    of0 = jnp.int32(0)
    vals, idxs, _, _, _, of = jax.lax.fori_loop(
        0, _K, sel_body, (vals, idxs, s[0], a[0], p0, of0))

    ci = jax.lax.broadcasted_iota(jnp.int32, (_BR, _NP), 1)

    def slow_path(_):
        def slow_body(j, carry):
            svals, sidxs = carry
            logits = l_ref[...]
            sm = jnp.max(logits, axis=1, keepdims=True)
            shit = logits == sm
            sidx = jnp.min(jnp.where(shit, ci, _BIGI), axis=1, keepdims=True)
            l_ref[...] = jnp.where(ci == sidx, _NEG, logits)
            svals = jnp.where(ki == j, sm, svals)
            sidxs = jnp.where(ki == j, sidx, sidxs)
            return svals, sidxs

        return jax.lax.fori_loop(
            0, _K, slow_body,
            (jnp.zeros((_BR, _K), jnp.float32), jnp.zeros((_BR, _K), jnp.int32)))

    vals, idxs = jax.lax.cond(
        of > 0, slow_path, lambda _: (vals, idxs), 0)

    e = jnp.exp(vals - vals[:, 0:1])
    w_ref[...] = e / jnp.sum(e, axis=1, keepdims=True)
    idx_ref[...] = idxs


def _build(X, W_head, b_head, W_tail, b_tail):
    Xp = jnp.pad(X, ((0, _NP - _N), (0, 0)))
    pb = _NP // _CW
    eh, et = pl.pallas_call(
        _proj_kernel,
        grid=(pb,),
        in_specs=[
            pl.BlockSpec((_CW, _DIM), lambda i: (i, 0)),
            pl.BlockSpec((_DIM, _DIM), lambda i: (0, 0)),
            pl.BlockSpec((1, _DIM), lambda i: (0, 0)),
            pl.BlockSpec((_DIM, _DIM), lambda i: (0, 0)),
            pl.BlockSpec((1, _DIM), lambda i: (0, 0)),
        ],
        out_specs=[
            pl.BlockSpec((_CW, _DIM), lambda i: (i, 0)),
            pl.BlockSpec((_CW, _DIM), lambda i: (i, 0)),
        ],
        out_shape=[
            jax.ShapeDtypeStruct((_NP, _DIM), jnp.float32),
            jax.ShapeDtypeStruct((_NP, _DIM), jnp.float32),
        ],
    )(Xp, W_head.reshape(_DIM, _DIM), b_head.reshape(1, _DIM),
      W_tail.reshape(_DIM, _DIM), b_tail.reshape(1, _DIM))

    nblk = _NP // _BR
    idxs, ws = pl.pallas_call(
        _topk_kernel,
        grid=(nblk,),
        in_specs=[
            pl.BlockSpec((_BR, _DIM), lambda i: (i, 0)),
            pl.BlockSpec((_NP, _DIM), lambda i: (0, 0)),
        ],
        out_specs=[
            pl.BlockSpec((_BR, _K), lambda i: (i, 0)),
            pl.BlockSpec((_BR, _K), lambda i: (i, 0)),
        ],
        out_shape=[
            jax.ShapeDtypeStruct((_NP, _K), jnp.int32),
            jax.ShapeDtypeStruct((_NP, _K), jnp.float32),
        ],
        scratch_shapes=[pltpu.VMEM((_BR, _NP), jnp.float32)],
    )(eh, et)
    return idxs[:_N], ws[:_N]


def kernel(X, W_head, b_head, W_tail, b_tail):
    idxs, ws = _build(X, W_head, b_head, W_tail, b_tail)
    n = X.shape[0]
    src = jnp.repeat(jnp.arange(n, dtype=jnp.int32), _K)
    edge_index = jnp.stack([src, idxs.reshape(-1)], axis=0)
    return edge_index, ws.reshape(-1)


# speedup vs baseline: 7.4323x; 1.0643x over previous
import jax
import jax.numpy as jnp
from jax.experimental import pallas as pl
from jax.experimental.pallas import tpu as pltpu

_DIM = 512
_N = 10000
_NP = 10240
_K = 32
_BR = 256
_CW = 1024
_NEG = -3.0e38
_BIGI = 2 ** 30


def _proj_kernel(x_ref, wh_ref, bh_ref, wt_ref, bt_ref, eh_ref, et_ref):
    x = x_ref[...]
    eh_ref[...] = jax.lax.dot_general(
        x, wh_ref[...], (((1,), (1,)), ((), ())),
        preferred_element_type=jnp.float32) + bh_ref[...]
    et_ref[...] = jax.lax.dot_general(
        x, wt_ref[...], (((1,), (1,)), ((), ())),
        preferred_element_type=jnp.float32) + bt_ref[...]


_D = 6


def _topk_kernel(eh_ref, et_ref, idx_ref, w_ref, l_ref):
    scale = _DIM ** -0.5
    eh = eh_ref[...]
    n_chunks = _NP // _CW
    n_groups = _NP // 128
    cci = jax.lax.broadcasted_iota(jnp.int32, (_BR, _CW), 1)

    def mm_body(c, _):
        et = et_ref[pl.ds(c * _CW, _CW), :]
        prod = jax.lax.dot_general(
            eh, et, (((1,), (1,)), ((), ())),
            preferred_element_type=jnp.float32) * scale
        prod = jnp.where(c * _CW + cci < _N, prod, _NEG)
        l_ref[:, pl.ds(c * _CW, _CW)] = prod
        return 0

    jax.lax.fori_loop(0, n_chunks, mm_body, 0)

    ki = jax.lax.broadcasted_iota(jnp.int32, (_BR, _K), 1)
    lane = jax.lax.broadcasted_iota(jnp.int32, (_BR, 128), 1)

    def ins_body(g, carry):
        s = list(carry[:_D])
        a = list(carry[_D:])
        x = l_ref[:, pl.ds(g * 128, 128)]
        ax = jnp.full((_BR, 128), 0, jnp.int32) + g
        for d in range(_D):
            c = x > s[d]
            s[d], x = jnp.where(c, x, s[d]), jnp.where(c, s[d], x)
            a[d], ax = jnp.where(c, ax, a[d]), jnp.where(c, a[d], ax)
        return tuple(s) + tuple(a)

    init = tuple(jnp.full((_BR, 128), _NEG, jnp.float32) for _ in range(_D)) \
        + tuple(jnp.zeros((_BR, 128), jnp.int32) for _ in range(_D))
    carry = jax.lax.fori_loop(0, n_groups, ins_body, init)

    cand = jnp.concatenate(carry[:_D], axis=1)
    cols = jnp.concatenate([carry[_D + d] * 128 + lane for d in range(_D)],
                           axis=1)

    def sel_body(j, carry):
        vals, idxs, cand = carry
        m = jnp.max(cand, axis=1, keepdims=True)
        hit = cand == m
        idx = jnp.min(jnp.where(hit, cols, _BIGI), axis=1, keepdims=True)
        cand = jnp.where(cols == idx, _NEG, cand)
        vals = jnp.where(ki == j, m, vals)
        idxs = jnp.where(ki == j, idx, idxs)
        return vals, idxs, cand

    vals = jnp.zeros((_BR, _K), jnp.float32)
    idxs = jnp.zeros((_BR, _K), jnp.int32)
    vals, idxs, cand = jax.lax.fori_loop(0, _K, sel_body, (vals, idxs, cand))

    of = jnp.max((cand[:, (_D - 1) * 128:] == _NEG).astype(jnp.int32))

    ci = jax.lax.broadcasted_iota(jnp.int32, (_BR, _NP), 1)

    def slow_path(_):
        def slow_body(j, carry):
            svals, sidxs = carry
            logits = l_ref[...]
            sm = jnp.max(logits, axis=1, keepdims=True)
            shit = logits == sm
            sidx = jnp.min(jnp.where(shit, ci, _BIGI), axis=1, keepdims=True)
            l_ref[...] = jnp.where(ci == sidx, _NEG, logits)
            svals = jnp.where(ki == j, sm, svals)
            sidxs = jnp.where(ki == j, sidx, sidxs)
            return svals, sidxs

        return jax.lax.fori_loop(
            0, _K, slow_body,
            (jnp.zeros((_BR, _K), jnp.float32), jnp.zeros((_BR, _K), jnp.int32)))

    vals, idxs = jax.lax.cond(
        of > 0, slow_path, lambda _: (vals, idxs), 0)

    e = jnp.exp(vals - vals[:, 0:1])
    w_ref[...] = e / jnp.sum(e, axis=1, keepdims=True)
    idx_ref[...] = idxs


def _build(X, W_head, b_head, W_tail, b_tail):
    Xp = jnp.pad(X, ((0, _NP - _N), (0, 0)))
    pb = _NP // _CW
    eh, et = pl.pallas_call(
        _proj_kernel,
        grid=(pb,),
        in_specs=[
            pl.BlockSpec((_CW, _DIM), lambda i: (i, 0)),
            pl.BlockSpec((_DIM, _DIM), lambda i: (0, 0)),
            pl.BlockSpec((1, _DIM), lambda i: (0, 0)),
            pl.BlockSpec((_DIM, _DIM), lambda i: (0, 0)),
            pl.BlockSpec((1, _DIM), lambda i: (0, 0)),
        ],
        out_specs=[
            pl.BlockSpec((_CW, _DIM), lambda i: (i, 0)),
            pl.BlockSpec((_CW, _DIM), lambda i: (i, 0)),
        ],
        out_shape=[
            jax.ShapeDtypeStruct((_NP, _DIM), jnp.float32),
            jax.ShapeDtypeStruct((_NP, _DIM), jnp.float32),
        ],
    )(Xp, W_head.reshape(_DIM, _DIM), b_head.reshape(1, _DIM),
      W_tail.reshape(_DIM, _DIM), b_tail.reshape(1, _DIM))

    nblk = _NP // _BR
    idxs, ws = pl.pallas_call(
        _topk_kernel,
        grid=(nblk,),
        in_specs=[
            pl.BlockSpec((_BR, _DIM), lambda i: (i, 0)),
            pl.BlockSpec((_NP, _DIM), lambda i: (0, 0)),
        ],
        out_specs=[
            pl.BlockSpec((_BR, _K), lambda i: (i, 0)),
            pl.BlockSpec((_BR, _K), lambda i: (i, 0)),
        ],
        out_shape=[
            jax.ShapeDtypeStruct((_NP, _K), jnp.int32),
            jax.ShapeDtypeStruct((_NP, _K), jnp.float32),
        ],
        scratch_shapes=[pltpu.VMEM((_BR, _NP), jnp.float32)],
    )(eh, et)
    return idxs[:_N], ws[:_N]


def kernel(X, W_head, b_head, W_tail, b_tail):
    idxs, ws = _build(X, W_head, b_head, W_tail, b_tail)
    n = X.shape[0]
    src = jnp.repeat(jnp.arange(n, dtype=jnp.int32), _K)
    edge_index = jnp.stack([src, idxs.reshape(-1)], axis=0)
    return edge_index, ws.reshape(-1)


# speedup vs baseline: 7.5997x; 1.0225x over previous
import jax
import jax.numpy as jnp
from jax.experimental import pallas as pl
from jax.experimental.pallas import tpu as pltpu

_DIM = 512
_N = 10000
_NP = 10240
_K = 32
_BR = 256
_CW = 1024
_NEG = -3.0e38
_BIGI = 2 ** 30


def _proj_kernel(x_ref, wh_ref, bh_ref, wt_ref, bt_ref, eh_ref, et_ref):
    x = x_ref[...]
    eh_ref[...] = jax.lax.dot_general(
        x, wh_ref[...], (((1,), (1,)), ((), ())),
        preferred_element_type=jnp.float32) + bh_ref[...]
    et_ref[...] = jax.lax.dot_general(
        x, wt_ref[...], (((1,), (1,)), ((), ())),
        preferred_element_type=jnp.float32) + bt_ref[...]


_D = 6


def _topk_kernel(eh_ref, et_ref, idx_ref, w_ref, l_ref):
    scale = _DIM ** -0.5
    eh = eh_ref[...]
    n_chunks = _NP // _CW
    n_groups = _NP // 128
    cci = jax.lax.broadcasted_iota(jnp.int32, (_BR, _CW), 1)

    def mm_body(c, _):
        et = et_ref[pl.ds(c * _CW, _CW), :]
        prod = jax.lax.dot_general(
            eh, et, (((1,), (1,)), ((), ())),
            preferred_element_type=jnp.float32) * scale
        prod = jnp.where(c * _CW + cci < _N, prod, _NEG)
        l_ref[:, pl.ds(c * _CW, _CW)] = prod
        return 0

    jax.lax.fori_loop(0, n_chunks, mm_body, 0)

    ki = jax.lax.broadcasted_iota(jnp.int32, (_BR, _K), 1)
    lane = jax.lax.broadcasted_iota(jnp.int32, (_BR, 128), 1)

    def ins_body(g, carry):
        s = list(carry[:_D])
        a = list(carry[_D:])
        x = l_ref[:, pl.ds(g * 128, 128)]
        ax = jnp.full((_BR, 128), 0, jnp.int32) + g
        for d in range(_D):
            c = x > s[d]
            s[d], x = jnp.where(c, x, s[d]), jnp.where(c, s[d], x)
            a[d], ax = jnp.where(c, ax, a[d]), jnp.where(c, a[d], ax)
        return tuple(s) + tuple(a)

    init = tuple(jnp.full((_BR, 128), _NEG, jnp.float32) for _ in range(_D)) \
        + tuple(jnp.zeros((_BR, 128), jnp.int32) for _ in range(_D))
    carry = jax.lax.fori_loop(0, n_groups, ins_body, init)

    cand = jnp.concatenate(
        [jnp.swapaxes(carry[d], 0, 1) for d in range(_D)], axis=0)
    cols = jnp.concatenate(
        [jnp.swapaxes(carry[_D + d] * 128 + lane, 0, 1) for d in range(_D)],
        axis=0)

    kit = jax.lax.broadcasted_iota(jnp.int32, (_K, _BR), 0)

    def sel_body(j, carry):
        vals, idxs, cand = carry
        m = jnp.max(cand, axis=0, keepdims=True)
        hit = cand == m
        idx = jnp.min(jnp.where(hit, cols, _BIGI), axis=0, keepdims=True)
        cand = jnp.where(cols == idx, _NEG, cand)
        vals = jnp.where(kit == j, m, vals)
        idxs = jnp.where(kit == j, idx, idxs)
        return vals, idxs, cand

    vals = jnp.zeros((_K, _BR), jnp.float32)
    idxs = jnp.zeros((_K, _BR), jnp.int32)
    vals, idxs, cand = jax.lax.fori_loop(0, _K, sel_body, (vals, idxs, cand))

    of = jnp.max((cand[(_D - 1) * 128:, :] == _NEG).astype(jnp.int32))
    vals = jnp.swapaxes(vals, 0, 1)
    idxs = jnp.swapaxes(idxs, 0, 1)

    ci = jax.lax.broadcasted_iota(jnp.int32, (_BR, _NP), 1)

    def slow_path(_):
        def slow_body(j, carry):
            svals, sidxs = carry
            logits = l_ref[...]
            sm = jnp.max(logits, axis=1, keepdims=True)
            shit = logits == sm
            sidx = jnp.min(jnp.where(shit, ci, _BIGI), axis=1, keepdims=True)
            l_ref[...] = jnp.where(ci == sidx, _NEG, logits)
            svals = jnp.where(ki == j, sm, svals)
            sidxs = jnp.where(ki == j, sidx, sidxs)
            return svals, sidxs

        return jax.lax.fori_loop(
            0, _K, slow_body,
            (jnp.zeros((_BR, _K), jnp.float32), jnp.zeros((_BR, _K), jnp.int32)))

    vals, idxs = jax.lax.cond(
        of > 0, slow_path, lambda _: (vals, idxs), 0)

    e = jnp.exp(vals - vals[:, 0:1])
    w_ref[...] = e / jnp.sum(e, axis=1, keepdims=True)
    idx_ref[...] = idxs


def _build(X, W_head, b_head, W_tail, b_tail):
    Xp = jnp.pad(X, ((0, _NP - _N), (0, 0)))
    pb = _NP // _CW
    eh, et = pl.pallas_call(
        _proj_kernel,
        grid=(pb,),
        in_specs=[
            pl.BlockSpec((_CW, _DIM), lambda i: (i, 0)),
            pl.BlockSpec((_DIM, _DIM), lambda i: (0, 0)),
            pl.BlockSpec((1, _DIM), lambda i: (0, 0)),
            pl.BlockSpec((_DIM, _DIM), lambda i: (0, 0)),
            pl.BlockSpec((1, _DIM), lambda i: (0, 0)),
        ],
        out_specs=[
            pl.BlockSpec((_CW, _DIM), lambda i: (i, 0)),
            pl.BlockSpec((_CW, _DIM), lambda i: (i, 0)),
        ],
        out_shape=[
            jax.ShapeDtypeStruct((_NP, _DIM), jnp.float32),
            jax.ShapeDtypeStruct((_NP, _DIM), jnp.float32),
        ],
    )(Xp, W_head.reshape(_DIM, _DIM), b_head.reshape(1, _DIM),
      W_tail.reshape(_DIM, _DIM), b_tail.reshape(1, _DIM))

    nblk = _NP // _BR
    idxs, ws = pl.pallas_call(
        _topk_kernel,
        grid=(nblk,),
        in_specs=[
            pl.BlockSpec((_BR, _DIM), lambda i: (i, 0)),
            pl.BlockSpec((_NP, _DIM), lambda i: (0, 0)),
        ],
        out_specs=[
            pl.BlockSpec((_BR, _K), lambda i: (i, 0)),
            pl.BlockSpec((_BR, _K), lambda i: (i, 0)),
        ],
        out_shape=[
            jax.ShapeDtypeStruct((_NP, _K), jnp.int32),
            jax.ShapeDtypeStruct((_NP, _K), jnp.float32),
        ],
        scratch_shapes=[pltpu.VMEM((_BR, _NP), jnp.float32)],
    )(eh, et)
    return idxs[:_N], ws[:_N]


def kernel(X, W_head, b_head, W_tail, b_tail):
    idxs, ws = _build(X, W_head, b_head, W_tail, b_tail)
    n = X.shape[0]
    src = jnp.repeat(jnp.arange(n, dtype=jnp.int32), _K)
    edge_index = jnp.stack([src, idxs.reshape(-1)], axis=0)
    return edge_index, ws.reshape(-1)
